# baseline (speedup 1.0000x reference)
import jax
import jax.numpy as jnp
from jax.experimental import pallas as pl
from jax.experimental.pallas import tpu as pltpu

_ROWS = 8


def _monotone_key(x):
    xi = jax.lax.bitcast_convert_type(x, jnp.int32)
    return xi ^ (jax.lax.shift_right_arithmetic(xi, 31) & jnp.int32(0x7FFFFFFF))


def _mid(lo, hi):
    return (lo & hi) + jax.lax.shift_right_arithmetic(lo ^ hi, 1)


def _row_body(k_ref, p_ref, x_ref, o_ref):
    kk = k_ref[0]
    pp = p_ref[0]
    x = x_ref[...]
    v = x.shape[1]

    key = _monotone_key(x)
    col = jax.lax.broadcasted_iota(jnp.int32, x.shape, 1)

    def rsum(a):
        return jnp.sum(a, axis=1, keepdims=True)

    def bs_topk(_, c):
        lo, hi = c
        mid = _mid(lo, hi)
        cnt = rsum((key > mid).astype(jnp.int32))
        pred = cnt < kk
        return jnp.where(pred, lo, mid + 1), jnp.where(pred, mid, hi)

    full = jnp.zeros((_ROWS, 1), jnp.int32)
    tk, _ = jax.lax.fori_loop(
        0, 32, bs_topk, (full + jnp.int32(-2147483648), full + jnp.int32(2147483647)))

    m = jnp.max(x, axis=1, keepdims=True)
    e = jnp.where(key >= tk, jnp.exp(x - m), 0.0)
    z = rsum(e)
    pz = pp * z
    kmax = jnp.max(key, axis=1, keepdims=True)

    def bs_topp(_, c):
        lo, hi = c
        mid = _mid(lo, hi)
        g = rsum(jnp.where(key > mid, e, 0.0))
        pred = g < pz
        return jnp.where(pred, lo, mid + 1), jnp.where(pred, mid, hi)

    ub, _ = jax.lax.fori_loop(0, 32, bs_topp, (tk, kmax))

    gv = rsum(jnp.where(key > ub, e, 0.0))
    qe = jnp.max(jnp.where(key == ub, e, 0.0), axis=1, keepdims=True)
    c_eq = rsum((key == ub).astype(jnp.int32))

    def bs_tie(_, c):
        lo, hi = c
        mid = (lo + hi) // 2
        r = rsum(jnp.where((key == ub) & (col > mid), 1.0, 0.0))
        pred = gv + r * qe < pz
        return jnp.where(pred, lo, mid + 1), jnp.where(pred, mid, hi)

    def tie_search():
        i1, _ = jax.lax.fori_loop(0, 17, bs_tie, (full, full + jnp.int32(v - 1)))
        return i1

    i1 = jax.lax.cond(jnp.any(c_eq > 1), tie_search, lambda: full)

    keep = (key > ub) | ((key == ub) & (col >= i1))
    o_ref[...] = jnp.where(keep, x, -jnp.inf)


def kernel(logits, k, p):
    bsz, v = logits.shape
    nb = bsz // _ROWS
    return pl.pallas_call(
        _row_body,
        grid=(nb,),
        in_specs=[
            pl.BlockSpec((1, _ROWS, 1), lambda b: (b, 0, 0)),
            pl.BlockSpec((1, _ROWS, 1), lambda b: (b, 0, 0)),
            pl.BlockSpec((_ROWS, v), lambda b: (b, 0)),
        ],
        out_specs=pl.BlockSpec((_ROWS, v), lambda b: (b, 0)),
        out_shape=jax.ShapeDtypeStruct((bsz, v), logits.dtype),
        compiler_params=pltpu.CompilerParams(
            dimension_semantics=("parallel",)),
    )(k.reshape(nb, _ROWS, 1), p.reshape(nb, _ROWS, 1), logits)
